# baseline (speedup 1.0000x reference)
import jax
import jax.numpy as jnp
from jax import lax
from jax.experimental import pallas as pl
from jax.experimental.pallas import tpu as pltpu
from jax.experimental.pallas import tpu_sc as plsc

N_USERS = 5000
N_ITEMS = 5000
N = N_USERS + N_ITEMS
E = 320000
D = 128
SLOPE = 0.2

NC = 2
NS = 16
NW = NC * NS
N_PAD = 10240
K = 80
NB = 128
EPW = NB * K
E_PAD = NW * EPW
PAD_DST = N_PAD - 1
RPT = N_PAD // NS
RC = 80

_mesh = plsc.VectorSubcoreMesh(core_axis_name="c", subcore_axis_name="s")


def _sc_pass(idx_hbm, dst_hbm, tab_hbm, zeros_hbm,
             p_out,
             gidx_v, didx_v, rows_v,
             acc, sem):
    c = lax.axis_index("c")
    s = lax.axis_index("s")
    wid = c * NS + s

    pltpu.sync_copy(zeros_hbm, rows_v)
    for i in range(RPT // RC):
        pltpu.sync_copy(rows_v, acc.at[pl.ds(s * RPT + i * RC, RC)])
    plsc.subcore_barrier()

    def body(j, carry):
        base = pl.multiple_of(wid * EPW + j * K, 8)
        pltpu.sync_copy(idx_hbm.at[pl.ds(base, K)], gidx_v)
        pltpu.sync_copy(dst_hbm.at[pl.ds(base, K)], didx_v)
        pltpu.async_copy(tab_hbm.at[gidx_v], rows_v, sem).wait()
        pltpu.sync_copy(rows_v, acc.at[didx_v], add=True)
        return carry
    lax.fori_loop(0, NB, body, 0)

    plsc.subcore_barrier()

    for i in range(RPT // RC):
        r0 = s * RPT + i * RC
        pltpu.sync_copy(acc.at[pl.ds(r0, RC)], rows_v)
        pltpu.sync_copy(rows_v, p_out.at[c, pl.ds(r0, RC)])


_pass = pl.kernel(
    _sc_pass,
    out_type=jax.ShapeDtypeStruct((NC, N_PAD, D), jnp.float32),
    mesh=_mesh,
    scratch_types=[
        pltpu.VMEM((K,), jnp.int32),
        pltpu.VMEM((K,), jnp.int32),
        pltpu.VMEM((K, D), jnp.float32),
        pltpu.VMEM_SHARED((N_PAD, D), jnp.float32),
        pltpu.SemaphoreType.DMA,
    ],
)


def _tc_layer_body(s0, s1, t0, t1, d0, d1, w, x_out, n_out):
    deg = d0[:, :1] + d1[:, :1]
    inv = 1.0 / jnp.maximum(deg, 1.0)
    agg = (s0[...] + s1[...] + t0[...] + t1[...]) * inv
    h = jnp.dot(agg, w[...], preferred_element_type=jnp.float32)
    x = jnp.where(h >= 0.0, h, SLOPE * h)
    nrm = jnp.sqrt(jnp.sum(x * x, axis=1, keepdims=True))
    x_out[...] = x
    n_out[...] = x / jnp.maximum(nrm, 1e-12)


_TC_B = 1024


def _tc_layer(s_part, t_part, deg_part, w):
    spec0 = pl.BlockSpec((_TC_B, D), lambda i: (i, 0))
    wspec0 = pl.BlockSpec((D, D), lambda i: (0, 0))
    return pl.pallas_call(
        _tc_layer_body,
        grid=(N_PAD // _TC_B,),
        in_specs=[spec0, spec0, spec0, spec0, spec0, spec0, wspec0],
        out_specs=[spec0, spec0],
        out_shape=[
            jax.ShapeDtypeStruct((N_PAD, D), jnp.float32),
            jax.ShapeDtypeStruct((N_PAD, D), jnp.float32),
        ],
    )(s_part[0], s_part[1], t_part[0], t_part[1],
      deg_part[0], deg_part[1], w)


def _pad_edges(arr, fill):
    pad = jnp.full((E_PAD - E,), fill, dtype=arr.dtype)
    return jnp.concatenate([arr, pad])


def kernel(edge_index, time_seq, user_embd, item_embd, time_table, W1, W2):
    src = _pad_edges(edge_index[0], 0)
    dst = _pad_edges(edge_index[1], PAD_DST)
    tim = _pad_edges(time_seq, 0)
    zidx = jnp.zeros((E_PAD,), jnp.int32)
    zeros = jnp.zeros((K, D), jnp.float32)
    ones_tab = jnp.ones((8, D), jnp.float32)

    x0 = jnp.concatenate([user_embd, item_embd], axis=0)
    x0 = jnp.pad(x0, ((0, N_PAD - N), (0, 0)))

    t_part = _pass(tim, dst, time_table, zeros)
    deg_part = _pass(zidx, dst, ones_tab, zeros)

    s1_part = _pass(src, dst, x0, zeros)
    x1, n1 = _tc_layer(s1_part, t_part, deg_part, W1)

    s2_part = _pass(src, dst, x1, zeros)
    _, n2 = _tc_layer(s2_part, t_part, deg_part, W2)

    user_out = jnp.concatenate(
        [user_embd, n1[:N_USERS], n2[:N_USERS]], axis=1)
    item_out = jnp.concatenate(
        [item_embd, n1[N_USERS:N], n2[N_USERS:N]], axis=1)
    return (user_out, item_out)


# speedup vs baseline: 8.1501x; 8.1501x over previous
import jax
import jax.numpy as jnp
from jax import lax
from jax.experimental import pallas as pl
from jax.experimental.pallas import tpu as pltpu
from jax.experimental.pallas import tpu_sc as plsc

N_USERS = 5000
N_ITEMS = 5000
N = N_USERS + N_ITEMS
E = 320000
D = 128
SLOPE = 0.2

NC = 2
NS = 16
NW = NC * NS
N_PAD = 10240
K = 128
NB = 80
G = 8
NG = NB // G
EPW = NB * K
E_PAD = NW * EPW
PAD_DST = N_PAD - 1
RPT = N_PAD // NS
RC = 128

_mesh = plsc.VectorSubcoreMesh(core_axis_name="c", subcore_axis_name="s")


def _zero_acc(zeros_hbm, stage_v, acc, s):
    pltpu.sync_copy(zeros_hbm, stage_v)
    for i in range(RPT // RC):
        pltpu.sync_copy(stage_v, acc.at[pl.ds(s * RPT + i * RC, RC)])


def _write_acc(p_out, stage_v, acc, c, s):
    for i in range(RPT // RC):
        r0 = s * RPT + i * RC
        pltpu.sync_copy(acc.at[pl.ds(r0, RC)], stage_v)
        pltpu.sync_copy(stage_v, p_out.at[c, pl.ds(r0, RC)])


def _sc_pass(idx_hbm, dst_hbm, tab_hbm, zeros_hbm,
             p_out,
             gidx_v, didx_v, rows_v,
             acc, sem0, sem1):
    c = lax.axis_index("c")
    s = lax.axis_index("s")
    wid = c * NS + s
    sems = (sem0, sem1)

    _zero_acc(zeros_hbm, rows_v.at[0], acc, s)
    plsc.subcore_barrier()

    def outer(g, carry):
        base = pl.multiple_of((wid * NB + g * G) * 1, 8)
        pltpu.sync_copy(idx_hbm.at[pl.ds(base, G)], gidx_v)
        pltpu.sync_copy(dst_hbm.at[pl.ds(base, G)], didx_v)
        pltpu.async_copy(tab_hbm.at[gidx_v.at[0]], rows_v.at[0], sems[0])
        for b in range(G):
            slot = b % 2
            pltpu.make_async_copy(
                tab_hbm.at[gidx_v.at[b]], rows_v.at[slot], sems[slot]).wait()
            if b + 1 < G:
                pltpu.async_copy(tab_hbm.at[gidx_v.at[b + 1]],
                                 rows_v.at[(b + 1) % 2], sems[(b + 1) % 2])
            pltpu.sync_copy(rows_v.at[slot], acc.at[didx_v.at[b]], add=True)
        return carry
    lax.fori_loop(0, NG, outer, 0)

    plsc.subcore_barrier()
    _write_acc(p_out, rows_v.at[0], acc, c, s)


def _sc_deg(dst_hbm, ones_hbm, zeros_hbm,
            p_out,
            didx_v, buf_v,
            acc):
    c = lax.axis_index("c")
    s = lax.axis_index("s")
    wid = c * NS + s

    _zero_acc(zeros_hbm, buf_v, acc, s)
    pltpu.sync_copy(ones_hbm, buf_v)
    plsc.subcore_barrier()

    def outer(g, carry):
        base = pl.multiple_of((wid * NB + g * G) * 1, 8)
        pltpu.sync_copy(dst_hbm.at[pl.ds(base, G)], didx_v)
        for b in range(G):
            pltpu.sync_copy(buf_v, acc.at[didx_v.at[b]], add=True)
        return carry
    lax.fori_loop(0, NG, outer, 0)

    plsc.subcore_barrier()
    _write_acc(p_out, buf_v, acc, c, s)


_pass = pl.kernel(
    _sc_pass,
    out_type=jax.ShapeDtypeStruct((NC, N_PAD, D), jnp.float32),
    mesh=_mesh,
    scratch_types=[
        pltpu.VMEM((G, K), jnp.int32),
        pltpu.VMEM((G, K), jnp.int32),
        pltpu.VMEM((2, K, D), jnp.float32),
        pltpu.VMEM_SHARED((N_PAD, D), jnp.float32),
        pltpu.SemaphoreType.DMA,
        pltpu.SemaphoreType.DMA,
    ],
)

_deg = pl.kernel(
    _sc_deg,
    out_type=jax.ShapeDtypeStruct((NC, N_PAD, D), jnp.float32),
    mesh=_mesh,
    scratch_types=[
        pltpu.VMEM((G, K), jnp.int32),
        pltpu.VMEM((K, D), jnp.float32),
        pltpu.VMEM_SHARED((N_PAD, D), jnp.float32),
    ],
)


def _tc_layer_body(s0, s1, t0, t1, d0, d1, w, x_out, n_out):
    deg = d0[:, :1] + d1[:, :1]
    inv = 1.0 / jnp.maximum(deg, 1.0)
    agg = (s0[...] + s1[...] + t0[...] + t1[...]) * inv
    h = jnp.dot(agg, w[...], preferred_element_type=jnp.float32)
    x = jnp.where(h >= 0.0, h, SLOPE * h)
    nrm = jnp.sqrt(jnp.sum(x * x, axis=1, keepdims=True))
    x_out[...] = x
    n_out[...] = x / jnp.maximum(nrm, 1e-12)


_TC_B = 1024


def _tc_layer(s_part, t_part, deg_part, w):
    spec0 = pl.BlockSpec((_TC_B, D), lambda i: (i, 0))
    wspec0 = pl.BlockSpec((D, D), lambda i: (0, 0))
    return pl.pallas_call(
        _tc_layer_body,
        grid=(N_PAD // _TC_B,),
        in_specs=[spec0, spec0, spec0, spec0, spec0, spec0, wspec0],
        out_specs=[spec0, spec0],
        out_shape=[
            jax.ShapeDtypeStruct((N_PAD, D), jnp.float32),
            jax.ShapeDtypeStruct((N_PAD, D), jnp.float32),
        ],
    )(s_part[0], s_part[1], t_part[0], t_part[1],
      deg_part[0], deg_part[1], w)


def _pad_edges(arr, fill):
    pad = jnp.full((E_PAD - E,), fill, dtype=arr.dtype)
    return jnp.concatenate([arr, pad]).reshape(NW * NB, K)


def kernel(edge_index, time_seq, user_embd, item_embd, time_table, W1, W2):
    src = _pad_edges(edge_index[0], 0)
    dst = _pad_edges(edge_index[1], PAD_DST)
    tim = _pad_edges(time_seq, 0)
    zeros = jnp.zeros((RC, D), jnp.float32)
    ones = jnp.ones((K, D), jnp.float32)

    x0 = jnp.concatenate([user_embd, item_embd], axis=0)
    x0 = jnp.pad(x0, ((0, N_PAD - N), (0, 0)))

    t_part = _pass(tim, dst, time_table, zeros)
    deg_part = _deg(dst, ones, zeros)

    s1_part = _pass(src, dst, x0, zeros)
    x1, n1 = _tc_layer(s1_part, t_part, deg_part, W1)

    s2_part = _pass(src, dst, x1, zeros)
    _, n2 = _tc_layer(s2_part, t_part, deg_part, W2)

    user_out = jnp.concatenate(
        [user_embd, n1[:N_USERS], n2[:N_USERS]], axis=1)
    item_out = jnp.concatenate(
        [item_embd, n1[N_USERS:N], n2[N_USERS:N]], axis=1)
    return (user_out, item_out)


# speedup vs baseline: 8.8414x; 1.0848x over previous
import jax
import jax.numpy as jnp
from jax import lax
from jax.experimental import pallas as pl
from jax.experimental.pallas import tpu as pltpu
from jax.experimental.pallas import tpu_sc as plsc

N_USERS = 5000
N_ITEMS = 5000
N = N_USERS + N_ITEMS
E = 320000
D = 128
SLOPE = 0.2

NC = 2
NS = 16
NW = NC * NS
N_PAD = 10240
K = 128
NB = 80
G = 8
NG = NB // G
EPW = NB * K
E_PAD = NW * EPW
PAD_DST = N_PAD - 1
RPT = N_PAD // NS
RC = 128

NB0 = 40
NB1 = 120
NG0 = NB0 // G
NG1 = NB1 // G
CORE1_BASE = NS * NB0

_mesh = plsc.VectorSubcoreMesh(core_axis_name="c", subcore_axis_name="s")


def _zero_acc(zeros_hbm, stage_v, acc, s):
    pltpu.sync_copy(zeros_hbm, stage_v)
    for i in range(RPT // RC):
        pltpu.sync_copy(stage_v, acc.at[pl.ds(s * RPT + i * RC, RC)])


def _write_acc(p_out, stage_v, acc, c, s):
    for i in range(RPT // RC):
        r0 = s * RPT + i * RC
        pltpu.sync_copy(acc.at[pl.ds(r0, RC)], stage_v)
        pltpu.sync_copy(stage_v, p_out.at[c, pl.ds(r0, RC)])


def _sc_pass(idx_hbm, dst_hbm, tab_hbm, zeros_hbm,
             p_out,
             gidx_v, didx_v, rows_v,
             acc, sem0, sem1):
    c = lax.axis_index("c")
    s = lax.axis_index("s")
    wid = c * NS + s
    sems = (sem0, sem1)

    _zero_acc(zeros_hbm, rows_v.at[0], acc, s)
    plsc.subcore_barrier()

    def outer(g, carry):
        base = pl.multiple_of((wid * NB + g * G) * 1, 8)
        pltpu.sync_copy(idx_hbm.at[pl.ds(base, G)], gidx_v)
        pltpu.sync_copy(dst_hbm.at[pl.ds(base, G)], didx_v)
        pltpu.async_copy(tab_hbm.at[gidx_v.at[0]], rows_v.at[0], sems[0])
        for b in range(G):
            slot = b % 2
            pltpu.make_async_copy(
                tab_hbm.at[gidx_v.at[b]], rows_v.at[slot], sems[slot]).wait()
            if b + 1 < G:
                pltpu.async_copy(tab_hbm.at[gidx_v.at[b + 1]],
                                 rows_v.at[(b + 1) % 2], sems[(b + 1) % 2])
            pltpu.sync_copy(rows_v.at[slot], acc.at[didx_v.at[b]], add=True)
        return carry
    lax.fori_loop(0, NG, outer, 0)

    plsc.subcore_barrier()
    _write_acc(p_out, rows_v.at[0], acc, c, s)


def _sc_deg(dst_hbm, ones_hbm, zeros_hbm,
            p_out,
            didx_v, buf_v,
            acc):
    c = lax.axis_index("c")
    s = lax.axis_index("s")
    wid = c * NS + s

    _zero_acc(zeros_hbm, buf_v, acc, s)
    pltpu.sync_copy(ones_hbm, buf_v)
    plsc.subcore_barrier()

    def outer(g, carry):
        base = pl.multiple_of((wid * NB + g * G) * 1, 8)
        pltpu.sync_copy(dst_hbm.at[pl.ds(base, G)], didx_v)
        for b in range(G):
            pltpu.sync_copy(buf_v, acc.at[didx_v.at[b]], add=True)
        return carry
    lax.fori_loop(0, NG, outer, 0)

    plsc.subcore_barrier()
    _write_acc(p_out, buf_v, acc, c, s)


_pass = pl.kernel(
    _sc_pass,
    out_type=jax.ShapeDtypeStruct((NC, N_PAD, D), jnp.float32),
    mesh=_mesh,
    scratch_types=[
        pltpu.VMEM((G, K), jnp.int32),
        pltpu.VMEM((G, K), jnp.int32),
        pltpu.VMEM((2, K, D), jnp.float32),
        pltpu.VMEM_SHARED((N_PAD, D), jnp.float32),
        pltpu.SemaphoreType.DMA,
        pltpu.SemaphoreType.DMA,
    ],
)

_deg = pl.kernel(
    _sc_deg,
    out_type=jax.ShapeDtypeStruct((NC, N_PAD, D), jnp.float32),
    mesh=_mesh,
    scratch_types=[
        pltpu.VMEM((G, K), jnp.int32),
        pltpu.VMEM((K, D), jnp.float32),
        pltpu.VMEM_SHARED((N_PAD, D), jnp.float32),
    ],
)


def _tc_layer_body(s0, s1, t0, t1, d0, d1, w, x_out, n_out):
    deg = d0[:, :1] + d1[:, :1]
    inv = 1.0 / jnp.maximum(deg, 1.0)
    agg = (s0[...] + s1[...] + t0[...] + t1[...]) * inv
    h = jnp.dot(agg, w[...], preferred_element_type=jnp.float32)
    x = jnp.where(h >= 0.0, h, SLOPE * h)
    nrm = jnp.sqrt(jnp.sum(x * x, axis=1, keepdims=True))
    x_out[...] = x
    n_out[...] = x / jnp.maximum(nrm, 1e-12)


_TC_B = 1024


def _tc_layer(s_part, t_part, deg_part, w):
    spec0 = pl.BlockSpec((_TC_B, D), lambda i: (i, 0))
    wspec0 = pl.BlockSpec((D, D), lambda i: (0, 0))
    return pl.pallas_call(
        _tc_layer_body,
        grid=(N_PAD // _TC_B,),
        in_specs=[spec0, spec0, spec0, spec0, spec0, spec0, wspec0],
        out_specs=[spec0, spec0],
        out_shape=[
            jax.ShapeDtypeStruct((N_PAD, D), jnp.float32),
            jax.ShapeDtypeStruct((N_PAD, D), jnp.float32),
        ],
    )(s_part[0], s_part[1], t_part[0], t_part[1],
      deg_part[0], deg_part[1], w)


def _pad_edges(arr, fill):
    pad = jnp.full((E_PAD - E,), fill, dtype=arr.dtype)
    return jnp.concatenate([arr, pad]).reshape(NW * NB, K)


def kernel(edge_index, time_seq, user_embd, item_embd, time_table, W1, W2):
    src = _pad_edges(edge_index[0], 0)
    dst = _pad_edges(edge_index[1], PAD_DST)
    tim = _pad_edges(time_seq, 0)
    zeros = jnp.zeros((RC, D), jnp.float32)
    ones = jnp.ones((K, D), jnp.float32)

    x0 = jnp.concatenate([user_embd, item_embd], axis=0)
    x0 = jnp.pad(x0, ((0, N_PAD - N), (0, 0)))

    t_part = _pass(tim, dst, time_table, zeros)
    deg_part = _deg(dst, ones, zeros)

    s1_part = _pass(src, dst, x0, zeros)
    x1, n1 = _tc_layer(s1_part, t_part, deg_part, W1)

    s2_part = _pass(src, dst, x1, zeros)
    _, n2 = _tc_layer(s2_part, t_part, deg_part, W2)

    user_out = jnp.concatenate(
        [user_embd, n1[:N_USERS], n2[:N_USERS]], axis=1)
    item_out = jnp.concatenate(
        [item_embd, n1[N_USERS:N], n2[N_USERS:N]], axis=1)
    return (user_out, item_out)
